# baseline (speedup 1.0000x reference)
import functools

import jax
import jax.numpy as jnp
from jax import lax
from jax.experimental import pallas as pl
from jax.experimental.pallas import tpu as pltpu
from jax.experimental.pallas import tpu_sc as plsc

NUM_CORES = 2
NUM_SUBCORES = 16
NW = NUM_CORES * NUM_SUBCORES
CHUNK = 128


@functools.partial(jax.jit, static_argnames=("n_chunks", "emb_dim"))
def _emb_lookup(x_split, table, n_chunks, emb_dim):
    b_per_w = n_chunks * CHUNK

    mesh = plsc.VectorSubcoreMesh(core_axis_name="c", subcore_axis_name="s")

    @functools.partial(
        pl.kernel,
        out_type=jax.ShapeDtypeStruct((NW * b_per_w, emb_dim), jnp.float32),
        mesh=mesh,
        scratch_types=[
            pltpu.VMEM((n_chunks, CHUNK), jnp.int32),
            pltpu.VMEM((CHUNK, emb_dim), jnp.float32),
            pltpu.SemaphoreType.DMA,
        ],
        compiler_params=pltpu.CompilerParams(use_tc_tiling_on_sc=False),
    )
    def emb_kernel(x_hbm, tab_hbm, out_hbm, idx_v, rows_v, sem):
        c = lax.axis_index("c")
        s = lax.axis_index("s")
        wid = s * NUM_CORES + c
        base = wid * b_per_w
        pltpu.sync_copy(x_hbm.at[wid], idx_v)

        def body(j, carry):
            pltpu.async_copy(tab_hbm.at[idx_v.at[j]], rows_v, sem).wait()
            pltpu.sync_copy(rows_v, out_hbm.at[pl.ds(base + j * CHUNK, CHUNK)])
            return carry

        lax.fori_loop(0, n_chunks, body, 0)

    return emb_kernel(x_split, table)


def kernel(x, table):
    batch, hist = x.shape
    vocab, emb_dim = table.shape
    total = batch * hist
    assert total % (NW * CHUNK) == 0
    n_chunks = total // (NW * CHUNK)
    x_split = x.reshape(NW, n_chunks, CHUNK).astype(jnp.int32)
    out = _emb_lookup(x_split, table, n_chunks, emb_dim)
    return out.reshape(batch, hist, emb_dim)


# speedup vs baseline: 1.0450x; 1.0450x over previous
import functools

import jax
import jax.numpy as jnp
from jax import lax
from jax.experimental import pallas as pl
from jax.experimental.pallas import tpu as pltpu
from jax.experimental.pallas import tpu_sc as plsc

NUM_CORES = 2
NUM_SUBCORES = 16
NW = NUM_CORES * NUM_SUBCORES
CHUNK = 128
G = 5
NBUF = 5


@functools.partial(jax.jit, static_argnames=("n_chunks", "emb_dim"))
def _emb_lookup(x_split, table, n_chunks, emb_dim):
    b_per_w = n_chunks * CHUNK
    n_groups = n_chunks // G

    mesh = plsc.VectorSubcoreMesh(core_axis_name="c", subcore_axis_name="s")

    @functools.partial(
        pl.kernel,
        out_type=jax.ShapeDtypeStruct((NW * b_per_w, emb_dim), jnp.float32),
        mesh=mesh,
        scratch_types=[
            pltpu.VMEM((n_chunks, CHUNK), jnp.int32),
            pltpu.VMEM((NBUF, G * CHUNK, emb_dim), jnp.float32),
            [pltpu.SemaphoreType.DMA] * NBUF,
            [pltpu.SemaphoreType.DMA] * NBUF,
        ],
        compiler_params=pltpu.CompilerParams(use_tc_tiling_on_sc=False),
    )
    def emb_kernel(x_hbm, tab_hbm, out_hbm, idx_v, rows_v, gsems, wsems):
        c = lax.axis_index("c")
        s = lax.axis_index("s")
        wid = s * NUM_CORES + c
        base = wid * b_per_w
        pltpu.sync_copy(x_hbm.at[wid], idx_v)

        def fire_group(k, b):
            return [
                pltpu.async_copy(
                    tab_hbm.at[idx_v.at[k * G + g]],
                    rows_v.at[b, pl.ds(g * CHUNK, CHUNK)],
                    gsems[b],
                )
                for g in range(G)
            ]

        descs = [fire_group(b, b) for b in range(NBUF)]
        for k in range(n_groups):
            b = k % NBUF
            for d in descs[b]:
                d.wait()
            wd = pltpu.async_copy(
                rows_v.at[b],
                out_hbm.at[pl.ds(base + k * G * CHUNK, G * CHUNK)],
                wsems[b],
            )
            wd.wait()
            nk = k + NBUF
            if nk < n_groups:
                descs[b] = fire_group(nk, b)

    return emb_kernel(x_split, table)


def kernel(x, table):
    batch, hist = x.shape
    vocab, emb_dim = table.shape
    total = batch * hist
    assert total % (NW * CHUNK * G) == 0
    n_chunks = total // (NW * CHUNK)
    x_split = x.reshape(NW, n_chunks, CHUNK).astype(jnp.int32)
    out = _emb_lookup(x_split, table, n_chunks, emb_dim)
    return out.reshape(batch, hist, emb_dim)


# speedup vs baseline: 1.0460x; 1.0010x over previous
import functools

import jax
import jax.numpy as jnp
from jax import lax
from jax.experimental import pallas as pl
from jax.experimental.pallas import tpu as pltpu
from jax.experimental.pallas import tpu_sc as plsc

NUM_CORES = 2
NUM_SUBCORES = 16
NW = NUM_CORES * NUM_SUBCORES
CHUNK = 128
G = 5
NBUF = 5


@functools.partial(jax.jit, static_argnames=("n_chunks", "vocab", "emb_dim"))
def _emb_lookup(x_flat, tab_flat, n_chunks, vocab, emb_dim):
    b_per_w = n_chunks * CHUNK
    n_groups = n_chunks // G

    mesh = plsc.VectorSubcoreMesh(core_axis_name="c", subcore_axis_name="s")

    @functools.partial(
        pl.kernel,
        out_type=jax.ShapeDtypeStruct((NW * b_per_w, emb_dim), jnp.float32),
        mesh=mesh,
        scratch_types=[
            pltpu.VMEM((b_per_w,), jnp.int32),
            pltpu.VMEM((NBUF, G * CHUNK, emb_dim), jnp.float32),
            [pltpu.SemaphoreType.DMA] * NBUF,
            [pltpu.SemaphoreType.DMA] * NBUF,
        ],
        compiler_params=pltpu.CompilerParams(use_tc_tiling_on_sc=False),
    )
    def emb_kernel(x_hbm, tab2d, out_hbm, idx_v, rows_v, gsems, wsems):
        c = lax.axis_index("c")
        s = lax.axis_index("s")
        wid = s * NUM_CORES + c
        base = wid * b_per_w
        pltpu.sync_copy(x_hbm.at[pl.ds(base, b_per_w)], idx_v)

        def fire_group(k, b):
            return [
                pltpu.async_copy(
                    tab2d.at[idx_v.at[pl.ds((k * G + g) * CHUNK, CHUNK)]],
                    rows_v.at[b, pl.ds(g * CHUNK, CHUNK)],
                    gsems[b],
                )
                for g in range(G)
            ]

        descs = [fire_group(b, b) for b in range(NBUF)]
        for k in range(n_groups):
            b = k % NBUF
            for d in descs[b]:
                d.wait()
            wd = pltpu.async_copy(
                rows_v.at[b],
                out_hbm.at[pl.ds(base + k * G * CHUNK, G * CHUNK)],
                wsems[b],
            )
            wd.wait()
            nk = k + NBUF
            if nk < n_groups:
                descs[b] = fire_group(nk, b)

    return emb_kernel(x_flat, tab_flat.reshape(vocab, emb_dim))


def kernel(x, table):
    batch, hist = x.shape
    vocab, emb_dim = table.shape
    total = batch * hist
    assert total % (NW * CHUNK * G) == 0
    n_chunks = total // (NW * CHUNK)
    x_flat = x.reshape(-1).astype(jnp.int32)
    tab_flat = table.reshape(-1)
    out = _emb_lookup(x_flat, tab_flat, n_chunks, vocab, emb_dim)
    return out.reshape(batch, hist, emb_dim)


# speedup vs baseline: 1.2193x; 1.1657x over previous
import functools

import jax
import jax.numpy as jnp
from jax import lax
from jax.experimental import pallas as pl
from jax.experimental.pallas import tpu as pltpu
from jax.experimental.pallas import tpu_sc as plsc

NUM_CORES = 2
NUM_SUBCORES = 16
NW = NUM_CORES * NUM_SUBCORES
BBLK = 128
NBUF = 5
L = 16


@functools.partial(jax.jit, static_argnames=("hist", "emb_dim"))
def _emb_lookup(x_t, table, hist, emb_dim):
    batch = x_t.shape[1]
    n_groups = hist // NBUF
    assert hist == n_groups * NBUF and batch == NW * BBLK

    mesh = plsc.VectorSubcoreMesh(core_axis_name="c", subcore_axis_name="s")

    @functools.partial(
        pl.kernel,
        out_type=jax.ShapeDtypeStruct((hist, emb_dim, batch), jnp.float32),
        mesh=mesh,
        scratch_types=[
            pltpu.VMEM((hist, BBLK), jnp.int32),
            pltpu.VMEM((NBUF, BBLK, emb_dim), jnp.float32),
            pltpu.VMEM((NBUF, emb_dim, BBLK), jnp.float32),
            pltpu.SemaphoreType.DMA,
            [pltpu.SemaphoreType.DMA] * NBUF,
            [pltpu.SemaphoreType.DMA] * NBUF,
        ],
        compiler_params=pltpu.CompilerParams(
            use_tc_tiling_on_sc=False, needs_layout_passes=False
        ),
    )
    def emb_kernel(x_hbm, tab_hbm, out_hbm, idx_v, rows_v, outt_v, isem,
                   gsems, wsems):
        c = lax.axis_index("c")
        s = lax.axis_index("s")
        wid = s * NUM_CORES + c
        b0 = wid * BBLK
        pltpu.async_copy(x_hbm.at[:, pl.ds(b0, BBLK)], idx_v, isem).wait()

        lane = jax.lax.iota(jnp.int32, L)
        e_lo = lane
        e_hi = lane + L

        def transpose_slot(b):
            for r in range(BBLK):
                col = jnp.full((L,), r, jnp.int32)
                v0 = rows_v[b, r, pl.ds(0, L)]
                v1 = rows_v[b, r, pl.ds(L, L)]
                plsc.store_scatter(outt_v.at[b], [e_lo, col], v0)
                plsc.store_scatter(outt_v.at[b], [e_hi, col], v1)

        def body(k, carry):
            h0 = k * NBUF
            for b in range(NBUF):
                pltpu.make_async_copy(
                    out_hbm.at[b, :, pl.ds(b0, BBLK)], outt_v.at[b], wsems[b]
                ).wait()
                pltpu.async_copy(
                    tab_hbm.at[idx_v.at[h0 + b]], rows_v.at[b], gsems[b]
                )
            for b in range(NBUF):
                pltpu.make_async_copy(
                    tab_hbm.at[idx_v.at[h0 + b]], rows_v.at[b], gsems[b]
                ).wait()
                transpose_slot(b)
                pltpu.async_copy(
                    outt_v.at[b],
                    out_hbm.at[h0 + b, :, pl.ds(b0, BBLK)],
                    wsems[b],
                )
            return carry

        for b in range(NBUF):
            pltpu.async_copy(
                outt_v.at[b], out_hbm.at[b, :, pl.ds(b0, BBLK)], wsems[b]
            )
        lax.fori_loop(0, n_groups, body, 0)
        for b in range(NBUF):
            pltpu.make_async_copy(
                out_hbm.at[b, :, pl.ds(b0, BBLK)], outt_v.at[b], wsems[b]
            ).wait()

    return emb_kernel(x_t, table)


def kernel(x, table):
    batch, hist = x.shape
    vocab, emb_dim = table.shape
    assert emb_dim == 2 * L and batch == NW * BBLK
    x_t = x.T.astype(jnp.int32)
    out = _emb_lookup(x_t, table, hist, emb_dim)
    return jnp.transpose(out, (2, 0, 1))
